# baseline (speedup 1.0000x reference)
import functools

import jax
import jax.numpy as jnp
from jax.experimental import pallas as pl
from jax.experimental.pallas import tpu as pltpu
from jax.experimental.pallas import tpu_sc as plsc

_N = 4096
_D_IN = 2048
_D_FEAT = 512
_K = 8
_RB = 256
_PB = 512
_NW = 32
_SC_ROWS = _N // _NW
_CAP = 64


def _proj_kernel(x_ref, w_ref, o_ref):
    o_ref[...] = jax.lax.dot_general(
        x_ref[...], w_ref[...], (((1,), (0,)), ((), ())),
        preferred_element_type=jnp.float32)


def _project(images, w):
    return pl.pallas_call(
        _proj_kernel,
        grid=(_N // _PB,),
        in_specs=[
            pl.BlockSpec((_PB, _D_IN), lambda i: (i, 0)),
            pl.BlockSpec((_D_IN, _D_FEAT), lambda i: (0, 0)),
        ],
        out_specs=pl.BlockSpec((_PB, _D_FEAT), lambda i: (i, 0)),
        out_shape=jax.ShapeDtypeStruct((_N, _D_FEAT), jnp.float32),
    )(images, w)


def _d2_block(pts_ref, sq_ref, i):
    pts = pts_ref[...]

    @pl.when(i == 0)
    def _():
        s = jnp.sum(pts * pts, axis=1, keepdims=True)
        sq_ref[...] = jax.lax.transpose(s, (1, 0))

    sq = sq_ref[...]
    q = pts_ref[pl.ds(i * _RB, _RB), :]
    qsq = jnp.sum(q * q, axis=1, keepdims=True)
    mm = jax.lax.dot_general(
        q, pts, (((1,), (1,)), ((), ())),
        preferred_element_type=jnp.float32)
    return qsq + sq - 2.0 * mm


def _knn_kernel(pts_ref, nbr_ref, sq_ref):
    i = pl.program_id(0)
    d2 = _d2_block(pts_ref, sq_ref, i)

    cols = jax.lax.broadcasted_iota(
        jnp.int32, (_RB, _N), 1).astype(jnp.float32)
    kcols = jax.lax.broadcasted_iota(jnp.int32, (_RB, _K), 1)
    nbr = jnp.zeros((_RB, _K), jnp.int32)
    big = jnp.float32(_N)
    for t in range(_K + 1):
        mv = jnp.min(d2, axis=1, keepdims=True)
        idx = jnp.min(jnp.where(d2 == mv, cols, big), axis=1,
                      keepdims=True)
        if t > 0:
            nbr = jnp.where(kcols == (t - 1), idx.astype(jnp.int32), nbr)
        d2 = jnp.where(cols == idx, jnp.inf, d2)
    nbr_ref[...] = nbr


def _knn_neighbors(pts):
    d = pts.shape[1]
    return pl.pallas_call(
        _knn_kernel,
        grid=(_N // _RB,),
        in_specs=[pl.BlockSpec((_N, d), lambda i: (0, 0))],
        out_specs=pl.BlockSpec((_RB, _K), lambda i: (i, 0)),
        out_shape=jax.ShapeDtypeStruct((_N, _K), jnp.int32),
        scratch_shapes=[pltpu.VMEM((1, _N), jnp.float32)],
    )(pts)


def _d2_kernel(pts_ref, d2_ref, sq_ref):
    i = pl.program_id(0)
    d2_ref[...] = _d2_block(pts_ref, sq_ref, i)


def _d2_matrix(pts):
    d = pts.shape[1]
    return pl.pallas_call(
        _d2_kernel,
        grid=(_N // _RB,),
        in_specs=[pl.BlockSpec((_N, d), lambda i: (0, 0))],
        out_specs=pl.BlockSpec((_RB, _N), lambda i: (i, 0)),
        out_shape=jax.ShapeDtypeStruct((_N, _N), jnp.float32),
        scratch_shapes=[pltpu.VMEM((1, _N), jnp.float32)],
    )(pts)


def _sc_topk_body(d2_hbm, out_hbm, row_v, cand_v, candi_v, res_v):
    c = jax.lax.axis_index("c")
    s = jax.lax.axis_index("s")
    wid = s * 2 + c
    base = wid * _SC_ROWS
    lane = jax.lax.iota(jnp.int32, 16)
    inf = jnp.full((16,), jnp.inf, jnp.float32)
    bigi = jnp.full((16,), jnp.int32(1 << 30), jnp.int32)

    def row_body(r, carry):
        pltpu.sync_copy(d2_hbm.at[base + r], row_v)

        def amin(j, m):
            return jnp.minimum(m, row_v[pl.ds(j * 16, 16)])
        m = jax.lax.fori_loop(0, _N // 16, amin, inf)
        skey, _ = plsc.sort_key_val(m, m)
        tval = jnp.min(jnp.where(lane == 8, skey, jnp.inf))

        for k in range(_CAP // 16):
            cand_v[pl.ds(k * 16, 16)] = inf
            candi_v[pl.ds(k * 16, 16)] = bigi

        def bcol(j, off):
            v = row_v[pl.ds(j * 16, 16)]
            vi = lane + j * 16
            msk = v <= tval
            mi32 = msk.astype(jnp.int32)
            pos = off + plsc.cumsum(mi32) - mi32
            st = jnp.logical_and(msk, pos < _CAP)
            plsc.store_scatter(cand_v, [pos], v, mask=st)
            plsc.store_scatter(candi_v, [pos], vi, mask=st)
            return off + plsc.all_reduce_population_count(msk)
        jax.lax.fori_loop(0, _N // 16, bcol, jnp.zeros((16,), jnp.int32))

        cvs = [cand_v[pl.ds(k * 16, 16)] for k in range(_CAP // 16)]
        cis = [candi_v[pl.ds(k * 16, 16)] for k in range(_CAP // 16)]
        out_vec = jnp.zeros((16,), jnp.int32)
        for t in range(_K + 1):
            mv = cvs[0]
            for k in range(1, len(cvs)):
                mv = jnp.minimum(mv, cvs[k])
            mval = jnp.min(mv)
            mi = bigi
            for k in range(len(cvs)):
                mi = jnp.minimum(mi, jnp.where(cvs[k] == mval, cis[k], bigi))
            midx = jnp.min(mi)
            out_vec = jnp.where(lane == t, midx, out_vec)
            for k in range(len(cvs)):
                cvs[k] = jnp.where(cis[k] == midx, jnp.inf, cvs[k])
        res_v[pl.ds(r * 16, 16)] = out_vec
        return carry

    jax.lax.fori_loop(0, _SC_ROWS, row_body, jnp.int32(0))
    pltpu.sync_copy(res_v, out_hbm.at[pl.ds(base * 16, _SC_ROWS * 16)])


def _sc_topk(d2):
    mesh = plsc.VectorSubcoreMesh(core_axis_name="c", subcore_axis_name="s")
    fn = functools.partial(
        pl.kernel,
        mesh=mesh,
        out_type=jax.ShapeDtypeStruct((_N * 16,), jnp.int32),
        scratch_types=[
            pltpu.VMEM((_N,), jnp.float32),
            pltpu.VMEM((_CAP,), jnp.float32),
            pltpu.VMEM((_CAP,), jnp.int32),
            pltpu.VMEM((_SC_ROWS * 16,), jnp.int32),
        ],
        compiler_params=pltpu.CompilerParams(needs_layout_passes=False),
    )(_sc_topk_body)
    return fn(d2)


def kernel(images, img_coords, W):
    features = _project(images, W)
    d2_spatial = _d2_matrix(img_coords)
    nbr_spatial16 = _sc_topk(d2_spatial).reshape(_N, 16)
    nbr_latent = _knn_neighbors(features)
    src = jnp.repeat(jnp.arange(_N, dtype=jnp.int32), _K)
    edge_spatial = jnp.stack(
        [src, nbr_spatial16[:, 1:_K + 1].reshape(-1)], axis=0)
    edge_latent = jnp.stack([src, nbr_latent.reshape(-1)], axis=0)
    return features, edge_spatial, edge_latent, img_coords


# speedup vs baseline: 1.1700x; 1.1700x over previous
import functools

import jax
import jax.numpy as jnp
from jax.experimental import pallas as pl
from jax.experimental.pallas import tpu as pltpu
from jax.experimental.pallas import tpu_sc as plsc

_N = 4096
_D_IN = 2048
_D_FEAT = 512
_K = 8
_RB = 256
_PB = 512
_NW = 32
_SC_ROWS = _N // _NW
_CAP = 64


def _proj_kernel(x_ref, w_ref, o_ref):
    o_ref[...] = jax.lax.dot_general(
        x_ref[...], w_ref[...], (((1,), (0,)), ((), ())),
        preferred_element_type=jnp.float32)


def _project(images, w):
    return pl.pallas_call(
        _proj_kernel,
        grid=(_N // _PB,),
        in_specs=[
            pl.BlockSpec((_PB, _D_IN), lambda i: (i, 0)),
            pl.BlockSpec((_D_IN, _D_FEAT), lambda i: (0, 0)),
        ],
        out_specs=pl.BlockSpec((_PB, _D_FEAT), lambda i: (i, 0)),
        out_shape=jax.ShapeDtypeStruct((_N, _D_FEAT), jnp.float32),
    )(images, w)


def _d2_block(pts_ref, sq_ref, i):
    pts = pts_ref[...]

    @pl.when(i == 0)
    def _():
        s = jnp.sum(pts * pts, axis=1, keepdims=True)
        sq_ref[...] = jax.lax.transpose(s, (1, 0))

    sq = sq_ref[...]
    q = pts_ref[pl.ds(i * _RB, _RB), :]
    qsq = jnp.sum(q * q, axis=1, keepdims=True)
    mm = jax.lax.dot_general(
        q, pts, (((1,), (1,)), ((), ())),
        preferred_element_type=jnp.float32)
    return qsq + sq - 2.0 * mm


def _knn_kernel(pts_ref, nbr_ref, sq_ref):
    i = pl.program_id(0)
    d2 = _d2_block(pts_ref, sq_ref, i)

    cols = jax.lax.broadcasted_iota(
        jnp.int32, (_RB, _N), 1).astype(jnp.float32)
    kcols = jax.lax.broadcasted_iota(jnp.int32, (_RB, _K), 1)
    nbr = jnp.zeros((_RB, _K), jnp.int32)
    big = jnp.float32(_N)
    for t in range(_K + 1):
        mv = jnp.min(d2, axis=1, keepdims=True)
        idx = jnp.min(jnp.where(d2 == mv, cols, big), axis=1,
                      keepdims=True)
        if t > 0:
            nbr = jnp.where(kcols == (t - 1), idx.astype(jnp.int32), nbr)
        d2 = jnp.where(cols == idx, jnp.inf, d2)
    nbr_ref[...] = nbr


def _knn_neighbors(pts):
    d = pts.shape[1]
    return pl.pallas_call(
        _knn_kernel,
        grid=(_N // _RB,),
        in_specs=[pl.BlockSpec((_N, d), lambda i: (0, 0))],
        out_specs=pl.BlockSpec((_RB, _K), lambda i: (i, 0)),
        out_shape=jax.ShapeDtypeStruct((_N, _K), jnp.int32),
        scratch_shapes=[pltpu.VMEM((1, _N), jnp.float32)],
    )(pts)


def _d2_kernel(pts_ref, d2_ref, sq_ref):
    i = pl.program_id(0)
    d2_ref[...] = _d2_block(pts_ref, sq_ref, i)


def _d2_matrix(pts):
    d = pts.shape[1]
    return pl.pallas_call(
        _d2_kernel,
        grid=(_N // _RB,),
        in_specs=[pl.BlockSpec((_N, d), lambda i: (0, 0))],
        out_specs=pl.BlockSpec((_RB, _N), lambda i: (i, 0)),
        out_shape=jax.ShapeDtypeStruct((_N, _N), jnp.float32),
        scratch_shapes=[pltpu.VMEM((1, _N), jnp.float32)],
    )(pts)


def _sc_topk_body(d2_hbm, out_hbm, row_v, cand_v, candi_v, res_v):
    c = jax.lax.axis_index("c")
    s = jax.lax.axis_index("s")
    wid = s * 2 + c
    base = wid * _SC_ROWS
    lane = jax.lax.iota(jnp.int32, 16)
    inf = jnp.full((16,), jnp.inf, jnp.float32)
    bigi = jnp.full((16,), jnp.int32(1 << 30), jnp.int32)

    def row_body(r, carry):
        pltpu.sync_copy(d2_hbm.at[base + r], row_v)

        def amin(j, m):
            for u in range(8):
                m = jnp.minimum(m, row_v[pl.ds((j * 8 + u) * 16, 16)])
            return m
        m = jax.lax.fori_loop(0, _N // 128, amin, inf)
        skey, _ = plsc.sort_key_val(m, m)
        tval = jnp.min(jnp.where(lane == 8, skey, jnp.inf))

        for k in range(_CAP // 16):
            cand_v[pl.ds(k * 16, 16)] = inf
            candi_v[pl.ds(k * 16, 16)] = bigi

        def bcol(j, off):
            for u in range(4):
                jj = j * 4 + u
                v = row_v[pl.ds(jj * 16, 16)]
                vi = lane + jj * 16
                msk = v <= tval
                mi32 = msk.astype(jnp.int32)
                pos = off + plsc.cumsum(mi32) - mi32
                st = jnp.logical_and(msk, pos < _CAP)
                plsc.store_scatter(cand_v, [pos], v, mask=st)
                plsc.store_scatter(candi_v, [pos], vi, mask=st)
                off = off + plsc.all_reduce_population_count(msk)
            return off
        jax.lax.fori_loop(0, _N // 64, bcol, jnp.zeros((16,), jnp.int32))

        cvs = [cand_v[pl.ds(k * 16, 16)] for k in range(_CAP // 16)]
        cis = [candi_v[pl.ds(k * 16, 16)] for k in range(_CAP // 16)]
        out_vec = jnp.zeros((16,), jnp.int32)
        for t in range(_K + 1):
            mv = cvs[0]
            for k in range(1, len(cvs)):
                mv = jnp.minimum(mv, cvs[k])
            mval = jnp.min(mv)
            mi = bigi
            for k in range(len(cvs)):
                mi = jnp.minimum(mi, jnp.where(cvs[k] == mval, cis[k], bigi))
            midx = jnp.min(mi)
            out_vec = jnp.where(lane == t, midx, out_vec)
            for k in range(len(cvs)):
                cvs[k] = jnp.where(cis[k] == midx, jnp.inf, cvs[k])
        res_v[pl.ds(r * 16, 16)] = out_vec
        return carry

    jax.lax.fori_loop(0, _SC_ROWS, row_body, jnp.int32(0))
    pltpu.sync_copy(res_v, out_hbm.at[pl.ds(base * 16, _SC_ROWS * 16)])


def _sc_topk(d2):
    mesh = plsc.VectorSubcoreMesh(core_axis_name="c", subcore_axis_name="s")
    fn = functools.partial(
        pl.kernel,
        mesh=mesh,
        out_type=jax.ShapeDtypeStruct((_N * 16,), jnp.int32),
        scratch_types=[
            pltpu.VMEM((_N,), jnp.float32),
            pltpu.VMEM((_CAP,), jnp.float32),
            pltpu.VMEM((_CAP,), jnp.int32),
            pltpu.VMEM((_SC_ROWS * 16,), jnp.int32),
        ],
        compiler_params=pltpu.CompilerParams(needs_layout_passes=False),
    )(_sc_topk_body)
    return fn(d2)


def kernel(images, img_coords, W):
    features = _project(images, W)
    d2_spatial = _d2_matrix(img_coords)
    nbr_spatial16 = _sc_topk(d2_spatial).reshape(_N, 16)
    nbr_latent = _knn_neighbors(features)
    src = jnp.repeat(jnp.arange(_N, dtype=jnp.int32), _K)
    edge_spatial = jnp.stack(
        [src, nbr_spatial16[:, 1:_K + 1].reshape(-1)], axis=0)
    edge_latent = jnp.stack([src, nbr_latent.reshape(-1)], axis=0)
    return features, edge_spatial, edge_latent, img_coords


# speedup vs baseline: 1.5541x; 1.3283x over previous
import functools

import jax
import jax.numpy as jnp
from jax.experimental import pallas as pl
from jax.experimental.pallas import tpu as pltpu
from jax.experimental.pallas import tpu_sc as plsc

_N = 4096
_D_IN = 2048
_D_FEAT = 512
_K = 8
_RB = 256
_PB = 512
_NW = 32
_SC_ROWS = _N // _NW
_CAPL = 16
_CAP = 16 * _CAPL


def _proj_kernel(x_ref, w_ref, o_ref):
    o_ref[...] = jax.lax.dot_general(
        x_ref[...], w_ref[...], (((1,), (0,)), ((), ())),
        preferred_element_type=jnp.float32)


def _project(images, w):
    return pl.pallas_call(
        _proj_kernel,
        grid=(_N // _PB,),
        in_specs=[
            pl.BlockSpec((_PB, _D_IN), lambda i: (i, 0)),
            pl.BlockSpec((_D_IN, _D_FEAT), lambda i: (0, 0)),
        ],
        out_specs=pl.BlockSpec((_PB, _D_FEAT), lambda i: (i, 0)),
        out_shape=jax.ShapeDtypeStruct((_N, _D_FEAT), jnp.float32),
    )(images, w)


def _d2_block(pts_ref, sq_ref, i):
    pts = pts_ref[...]

    @pl.when(i == 0)
    def _():
        s = jnp.sum(pts * pts, axis=1, keepdims=True)
        sq_ref[...] = jax.lax.transpose(s, (1, 0))

    sq = sq_ref[...]
    q = pts_ref[pl.ds(i * _RB, _RB), :]
    qsq = jnp.sum(q * q, axis=1, keepdims=True)
    mm = jax.lax.dot_general(
        q, pts, (((1,), (1,)), ((), ())),
        preferred_element_type=jnp.float32)
    return qsq + sq - 2.0 * mm


def _knn_kernel(pts_ref, nbr_ref, sq_ref):
    i = pl.program_id(0)
    d2 = _d2_block(pts_ref, sq_ref, i)

    cols = jax.lax.broadcasted_iota(
        jnp.int32, (_RB, _N), 1).astype(jnp.float32)
    kcols = jax.lax.broadcasted_iota(jnp.int32, (_RB, _K), 1)
    nbr = jnp.zeros((_RB, _K), jnp.int32)
    big = jnp.float32(_N)
    for t in range(_K + 1):
        mv = jnp.min(d2, axis=1, keepdims=True)
        idx = jnp.min(jnp.where(d2 == mv, cols, big), axis=1,
                      keepdims=True)
        if t > 0:
            nbr = jnp.where(kcols == (t - 1), idx.astype(jnp.int32), nbr)
        d2 = jnp.where(cols == idx, jnp.inf, d2)
    nbr_ref[...] = nbr


def _knn_neighbors(pts):
    d = pts.shape[1]
    return pl.pallas_call(
        _knn_kernel,
        grid=(_N // _RB,),
        in_specs=[pl.BlockSpec((_N, d), lambda i: (0, 0))],
        out_specs=pl.BlockSpec((_RB, _K), lambda i: (i, 0)),
        out_shape=jax.ShapeDtypeStruct((_N, _K), jnp.int32),
        scratch_shapes=[pltpu.VMEM((1, _N), jnp.float32)],
    )(pts)


def _d2_kernel(pts_ref, d2_ref, sq_ref):
    i = pl.program_id(0)
    d2_ref[...] = _d2_block(pts_ref, sq_ref, i)


def _d2_matrix(pts):
    d = pts.shape[1]
    return pl.pallas_call(
        _d2_kernel,
        grid=(_N // _RB,),
        in_specs=[pl.BlockSpec((_N, d), lambda i: (0, 0))],
        out_specs=pl.BlockSpec((_RB, _N), lambda i: (i, 0)),
        out_shape=jax.ShapeDtypeStruct((_N, _N), jnp.float32),
        scratch_shapes=[pltpu.VMEM((1, _N), jnp.float32)],
    )(pts)


def _sc_topk_body(d2_hbm, out_hbm, row_v, cand_v, candi_v, res_v):
    c = jax.lax.axis_index("c")
    s = jax.lax.axis_index("s")
    wid = s * 2 + c
    base = wid * _SC_ROWS
    lane = jax.lax.iota(jnp.int32, 16)
    inf = jnp.full((16,), jnp.inf, jnp.float32)
    bigi = jnp.full((16,), jnp.int32(1 << 30), jnp.int32)

    def row_body(r, carry):
        pltpu.sync_copy(d2_hbm.at[base + r], row_v)

        def amin(j, m):
            for u in range(8):
                m = jnp.minimum(m, row_v[pl.ds((j * 8 + u) * 16, 16)])
            return m
        m = jax.lax.fori_loop(0, _N // 128, amin, inf)
        skey, _ = plsc.sort_key_val(m, m)
        tval = jnp.min(jnp.where(lane == 8, skey, jnp.inf))

        for k in range(_CAP // 16):
            cand_v[pl.ds(k * 16, 16)] = inf
            candi_v[pl.ds(k * 16, 16)] = bigi
        lane_base = lane * _CAPL

        def bcol(j, cnt):
            for u in range(4):
                jj = j * 4 + u
                v = row_v[pl.ds(jj * 16, 16)]
                vi = lane + jj * 16
                msk = v <= tval
                st = jnp.logical_and(msk, cnt < _CAPL)
                pos = lane_base + cnt
                plsc.store_scatter(cand_v, [pos], v, mask=st)
                plsc.store_scatter(candi_v, [pos], vi, mask=st)
                cnt = cnt + msk.astype(jnp.int32)
            return cnt
        jax.lax.fori_loop(0, _N // 64, bcol, jnp.zeros((16,), jnp.int32))

        cvs = [cand_v[pl.ds(k * 16, 16)] for k in range(_CAP // 16)]
        cis = [candi_v[pl.ds(k * 16, 16)] for k in range(_CAP // 16)]
        out_vec = jnp.zeros((16,), jnp.int32)
        for t in range(_K + 1):
            mv = cvs[0]
            for k in range(1, len(cvs)):
                mv = jnp.minimum(mv, cvs[k])
            mval = jnp.min(mv)
            mi = bigi
            for k in range(len(cvs)):
                mi = jnp.minimum(mi, jnp.where(cvs[k] == mval, cis[k], bigi))
            midx = jnp.min(mi)
            out_vec = jnp.where(lane == t, midx, out_vec)
            for k in range(len(cvs)):
                cvs[k] = jnp.where(cis[k] == midx, jnp.inf, cvs[k])
        res_v[pl.ds(r * 16, 16)] = out_vec
        return carry

    jax.lax.fori_loop(0, _SC_ROWS, row_body, jnp.int32(0))
    pltpu.sync_copy(res_v, out_hbm.at[pl.ds(base * 16, _SC_ROWS * 16)])


def _sc_topk(d2):
    mesh = plsc.VectorSubcoreMesh(core_axis_name="c", subcore_axis_name="s")
    fn = functools.partial(
        pl.kernel,
        mesh=mesh,
        out_type=jax.ShapeDtypeStruct((_N * 16,), jnp.int32),
        scratch_types=[
            pltpu.VMEM((_N,), jnp.float32),
            pltpu.VMEM((_CAP,), jnp.float32),
            pltpu.VMEM((_CAP,), jnp.int32),
            pltpu.VMEM((_SC_ROWS * 16,), jnp.int32),
        ],
        compiler_params=pltpu.CompilerParams(needs_layout_passes=False),
    )(_sc_topk_body)
    return fn(d2)


def kernel(images, img_coords, W):
    features = _project(images, W)
    d2_spatial = _d2_matrix(img_coords)
    nbr_spatial16 = _sc_topk(d2_spatial).reshape(_N, 16)
    nbr_latent = _knn_neighbors(features)
    src = jnp.repeat(jnp.arange(_N, dtype=jnp.int32), _K)
    edge_spatial = jnp.stack(
        [src, nbr_spatial16[:, 1:_K + 1].reshape(-1)], axis=0)
    edge_latent = jnp.stack([src, nbr_latent.reshape(-1)], axis=0)
    return features, edge_spatial, edge_latent, img_coords


# speedup vs baseline: 2.6258x; 1.6896x over previous
import functools

import jax
import jax.numpy as jnp
from jax.experimental import pallas as pl
from jax.experimental.pallas import tpu as pltpu
from jax.experimental.pallas import tpu_sc as plsc

_N = 4096
_D_IN = 2048
_D_FEAT = 512
_K = 8
_RB = 256
_PB = 512
_NW = 32
_SC_N = 2304
_SC_ROWS = _SC_N // _NW
_CAPL = 16
_CAP = 16 * _CAPL


def _proj_kernel(x_ref, w_ref, o_ref):
    o_ref[...] = jax.lax.dot_general(
        x_ref[...], w_ref[...], (((1,), (0,)), ((), ())),
        preferred_element_type=jnp.float32)


def _project(images, w):
    return pl.pallas_call(
        _proj_kernel,
        grid=(_N // _PB,),
        in_specs=[
            pl.BlockSpec((_PB, _D_IN), lambda i: (i, 0)),
            pl.BlockSpec((_D_IN, _D_FEAT), lambda i: (0, 0)),
        ],
        out_specs=pl.BlockSpec((_PB, _D_FEAT), lambda i: (i, 0)),
        out_shape=jax.ShapeDtypeStruct((_N, _D_FEAT), jnp.float32),
    )(images, w)


def _d2_block(pts_ref, sq_ref, i, *, first=None):
    pts = pts_ref[...]
    if first is None:
        first = i == 0

    @pl.when(first)
    def _():
        s = jnp.sum(pts * pts, axis=1, keepdims=True)
        sq_ref[...] = jax.lax.transpose(s, (1, 0))

    sq = sq_ref[...]
    q = pts_ref[pl.ds(i * _RB, _RB), :]
    qsq = jnp.sum(q * q, axis=1, keepdims=True)
    mm = jax.lax.dot_general(
        q, pts, (((1,), (1,)), ((), ())),
        preferred_element_type=jnp.float32)
    return qsq + sq - 2.0 * mm


def _knn_kernel(pts_ref, nbr_ref, sq_ref, *, block_offset=0):
    i = pl.program_id(0) + block_offset
    d2 = _d2_block(pts_ref, sq_ref, i, first=pl.program_id(0) == 0)

    cols = jax.lax.broadcasted_iota(
        jnp.int32, (_RB, _N), 1).astype(jnp.float32)
    kcols = jax.lax.broadcasted_iota(jnp.int32, (_RB, _K), 1)
    nbr = jnp.zeros((_RB, _K), jnp.int32)
    big = jnp.float32(_N)
    for t in range(_K + 1):
        mv = jnp.min(d2, axis=1, keepdims=True)
        idx = jnp.min(jnp.where(d2 == mv, cols, big), axis=1,
                      keepdims=True)
        if t > 0:
            nbr = jnp.where(kcols == (t - 1), idx.astype(jnp.int32), nbr)
        d2 = jnp.where(cols == idx, jnp.inf, d2)
    nbr_ref[...] = nbr


def _knn_neighbors(pts, block_offset=0):
    d = pts.shape[1]
    nblk = _N // _RB - block_offset
    return pl.pallas_call(
        functools.partial(_knn_kernel, block_offset=block_offset),
        grid=(nblk,),
        in_specs=[pl.BlockSpec((_N, d), lambda i: (0, 0))],
        out_specs=pl.BlockSpec((_RB, _K), lambda i: (i, 0)),
        out_shape=jax.ShapeDtypeStruct((nblk * _RB, _K), jnp.int32),
        scratch_shapes=[pltpu.VMEM((1, _N), jnp.float32)],
    )(pts)


def _d2_kernel(pts_ref, d2_ref, sq_ref):
    i = pl.program_id(0)
    d2_ref[...] = _d2_block(pts_ref, sq_ref, i)


def _d2_matrix(pts, nrows):
    d = pts.shape[1]
    return pl.pallas_call(
        _d2_kernel,
        grid=(nrows // _RB,),
        in_specs=[pl.BlockSpec((_N, d), lambda i: (0, 0))],
        out_specs=pl.BlockSpec((_RB, _N), lambda i: (i, 0)),
        out_shape=jax.ShapeDtypeStruct((nrows, _N), jnp.float32),
        scratch_shapes=[pltpu.VMEM((1, _N), jnp.float32)],
    )(pts)


def _sc_topk_body(d2_hbm, out_hbm, row_v, cand_v, candi_v, res_v):
    c = jax.lax.axis_index("c")
    s = jax.lax.axis_index("s")
    wid = s * 2 + c
    base = wid * _SC_ROWS
    lane = jax.lax.iota(jnp.int32, 16)
    inf = jnp.full((16,), jnp.inf, jnp.float32)
    bigi = jnp.full((16,), jnp.int32(1 << 30), jnp.int32)

    def row_body(r, carry):
        pltpu.sync_copy(d2_hbm.at[base + r], row_v)

        def amin(j, m):
            for u in range(8):
                m = jnp.minimum(m, row_v[pl.ds((j * 8 + u) * 16, 16)])
            return m
        m = jax.lax.fori_loop(0, _N // 128, amin, inf)
        skey, _ = plsc.sort_key_val(m, m)
        tval = jnp.min(jnp.where(lane == 8, skey, jnp.inf))

        for k in range(_CAP // 16):
            cand_v[pl.ds(k * 16, 16)] = inf
            candi_v[pl.ds(k * 16, 16)] = bigi
        lane_base = lane * _CAPL

        def bcol(j, cnt):
            for u in range(4):
                jj = j * 4 + u
                v = row_v[pl.ds(jj * 16, 16)]
                vi = lane + jj * 16
                msk = v <= tval
                st = jnp.logical_and(msk, cnt < _CAPL)
                pos = lane_base + cnt
                plsc.store_scatter(cand_v, [pos], v, mask=st)
                plsc.store_scatter(candi_v, [pos], vi, mask=st)
                cnt = cnt + msk.astype(jnp.int32)
            return cnt
        jax.lax.fori_loop(0, _N // 64, bcol, jnp.zeros((16,), jnp.int32))

        cvs = [cand_v[pl.ds(k * 16, 16)] for k in range(_CAP // 16)]
        cis = [candi_v[pl.ds(k * 16, 16)] for k in range(_CAP // 16)]
        out_vec = jnp.zeros((16,), jnp.int32)
        for t in range(_K + 1):
            mv = cvs[0]
            for k in range(1, len(cvs)):
                mv = jnp.minimum(mv, cvs[k])
            mval = jnp.min(mv)
            mi = bigi
            for k in range(len(cvs)):
                mi = jnp.minimum(mi, jnp.where(cvs[k] == mval, cis[k], bigi))
            midx = jnp.min(mi)
            out_vec = jnp.where(lane == t, midx, out_vec)
            for k in range(len(cvs)):
                cvs[k] = jnp.where(cis[k] == midx, jnp.inf, cvs[k])
        res_v[pl.ds(r * 16, 16)] = out_vec
        return carry

    jax.lax.fori_loop(0, _SC_ROWS, row_body, jnp.int32(0))
    pltpu.sync_copy(res_v, out_hbm.at[pl.ds(base * 16, _SC_ROWS * 16)])


def _sc_topk(d2):
    mesh = plsc.VectorSubcoreMesh(core_axis_name="c", subcore_axis_name="s")
    fn = functools.partial(
        pl.kernel,
        mesh=mesh,
        out_type=jax.ShapeDtypeStruct((_SC_N * 16,), jnp.int32),
        scratch_types=[
            pltpu.VMEM((_N,), jnp.float32),
            pltpu.VMEM((_CAP,), jnp.float32),
            pltpu.VMEM((_CAP,), jnp.int32),
            pltpu.VMEM((_SC_ROWS * 16,), jnp.int32),
        ],
        compiler_params=pltpu.CompilerParams(needs_layout_passes=False),
    )(_sc_topk_body)
    return fn(d2)


def kernel(images, img_coords, W):
    features = _project(images, W)
    d2_spatial = _d2_matrix(img_coords, _SC_N)
    nbr_sc16 = _sc_topk(d2_spatial).reshape(_SC_N, 16)
    nbr_latent = _knn_neighbors(features)
    nbr_sp_tail = _knn_neighbors(img_coords, block_offset=_SC_N // _RB)
    nbr_spatial = jnp.concatenate(
        [nbr_sc16[:, 1:_K + 1], nbr_sp_tail], axis=0)
    src = jnp.repeat(jnp.arange(_N, dtype=jnp.int32), _K)
    edge_spatial = jnp.stack([src, nbr_spatial.reshape(-1)], axis=0)
    edge_latent = jnp.stack([src, nbr_latent.reshape(-1)], axis=0)
    return features, edge_spatial, edge_latent, img_coords


# speedup vs baseline: 2.6375x; 1.0044x over previous
import functools

import jax
import jax.numpy as jnp
from jax.experimental import pallas as pl
from jax.experimental.pallas import tpu as pltpu
from jax.experimental.pallas import tpu_sc as plsc

_N = 4096
_D_IN = 2048
_D_FEAT = 512
_K = 8
_RB = 256
_PB = 512
_NW = 32
_SC_N = 2048
_SC_ROWS = _SC_N // _NW
_CAPL = 16
_CAP = 16 * _CAPL


def _proj_kernel(x_ref, w_ref, o_ref):
    o_ref[...] = jax.lax.dot_general(
        x_ref[...], w_ref[...], (((1,), (0,)), ((), ())),
        preferred_element_type=jnp.float32)


def _project(images, w):
    return pl.pallas_call(
        _proj_kernel,
        grid=(_N // _PB,),
        in_specs=[
            pl.BlockSpec((_PB, _D_IN), lambda i: (i, 0)),
            pl.BlockSpec((_D_IN, _D_FEAT), lambda i: (0, 0)),
        ],
        out_specs=pl.BlockSpec((_PB, _D_FEAT), lambda i: (i, 0)),
        out_shape=jax.ShapeDtypeStruct((_N, _D_FEAT), jnp.float32),
    )(images, w)


def _d2_block(pts_ref, sq_ref, i, *, first=None):
    pts = pts_ref[...]
    if first is None:
        first = i == 0

    @pl.when(first)
    def _():
        s = jnp.sum(pts * pts, axis=1, keepdims=True)
        sq_ref[...] = jax.lax.transpose(s, (1, 0))

    sq = sq_ref[...]
    q = pts_ref[pl.ds(i * _RB, _RB), :]
    qsq = jnp.sum(q * q, axis=1, keepdims=True)
    mm = jax.lax.dot_general(
        q, pts, (((1,), (1,)), ((), ())),
        preferred_element_type=jnp.float32)
    return qsq + sq - 2.0 * mm


def _knn_kernel(pts_ref, nbr_ref, sq_ref, *, block_offset=0):
    i = pl.program_id(0) + block_offset
    d2 = _d2_block(pts_ref, sq_ref, i, first=pl.program_id(0) == 0)

    cols = jax.lax.broadcasted_iota(
        jnp.int32, (_RB, _N), 1).astype(jnp.float32)
    kcols = jax.lax.broadcasted_iota(jnp.int32, (_RB, _K), 1)
    nbr = jnp.zeros((_RB, _K), jnp.int32)
    big = jnp.float32(_N)
    for t in range(_K + 1):
        mv = jnp.min(d2, axis=1, keepdims=True)
        idx = jnp.min(jnp.where(d2 == mv, cols, big), axis=1,
                      keepdims=True)
        if t > 0:
            nbr = jnp.where(kcols == (t - 1), idx.astype(jnp.int32), nbr)
        d2 = jnp.where(cols == idx, jnp.inf, d2)
    nbr_ref[...] = nbr


def _knn_neighbors(pts, block_offset=0):
    d = pts.shape[1]
    nblk = _N // _RB - block_offset
    return pl.pallas_call(
        functools.partial(_knn_kernel, block_offset=block_offset),
        grid=(nblk,),
        in_specs=[pl.BlockSpec((_N, d), lambda i: (0, 0))],
        out_specs=pl.BlockSpec((_RB, _K), lambda i: (i, 0)),
        out_shape=jax.ShapeDtypeStruct((nblk * _RB, _K), jnp.int32),
        scratch_shapes=[pltpu.VMEM((1, _N), jnp.float32)],
    )(pts)


def _d2_kernel(pts_ref, d2_ref, sq_ref):
    i = pl.program_id(0)
    d2_ref[...] = _d2_block(pts_ref, sq_ref, i)


def _d2_matrix(pts, nrows):
    d = pts.shape[1]
    return pl.pallas_call(
        _d2_kernel,
        grid=(nrows // _RB,),
        in_specs=[pl.BlockSpec((_N, d), lambda i: (0, 0))],
        out_specs=pl.BlockSpec((_RB, _N), lambda i: (i, 0)),
        out_shape=jax.ShapeDtypeStruct((nrows, _N), jnp.float32),
        scratch_shapes=[pltpu.VMEM((1, _N), jnp.float32)],
    )(pts)


def _sc_topk_body(d2_hbm, out_hbm, row_v, cand_v, candi_v, res_v):
    c = jax.lax.axis_index("c")
    s = jax.lax.axis_index("s")
    wid = s * 2 + c
    base = wid * _SC_ROWS
    lane = jax.lax.iota(jnp.int32, 16)
    inf = jnp.full((16,), jnp.inf, jnp.float32)
    bigi = jnp.full((16,), jnp.int32(1 << 30), jnp.int32)

    def row_body(r, carry):
        pltpu.sync_copy(d2_hbm.at[base + r], row_v)

        def amin(j, m):
            for u in range(8):
                m = jnp.minimum(m, row_v[pl.ds((j * 8 + u) * 16, 16)])
            return m
        m = jax.lax.fori_loop(0, _N // 128, amin, inf)
        skey, _ = plsc.sort_key_val(m, m)
        tval = jnp.min(jnp.where(lane == 8, skey, jnp.inf))

        for k in range(_CAP // 16):
            cand_v[pl.ds(k * 16, 16)] = inf
            candi_v[pl.ds(k * 16, 16)] = bigi
        lane_base = lane * _CAPL

        def bcol(j, cnt):
            for u in range(4):
                jj = j * 4 + u
                v = row_v[pl.ds(jj * 16, 16)]
                vi = lane + jj * 16
                msk = v <= tval
                st = jnp.logical_and(msk, cnt < _CAPL)
                pos = lane_base + cnt
                plsc.store_scatter(cand_v, [pos], v, mask=st)
                plsc.store_scatter(candi_v, [pos], vi, mask=st)
                cnt = cnt + msk.astype(jnp.int32)
            return cnt
        jax.lax.fori_loop(0, _N // 64, bcol, jnp.zeros((16,), jnp.int32))

        cvs = [cand_v[pl.ds(k * 16, 16)] for k in range(_CAP // 16)]
        cis = [candi_v[pl.ds(k * 16, 16)] for k in range(_CAP // 16)]
        out_vec = jnp.zeros((16,), jnp.int32)
        for t in range(_K + 1):
            mv = cvs[0]
            for k in range(1, len(cvs)):
                mv = jnp.minimum(mv, cvs[k])
            mval = jnp.min(mv)
            mi = bigi
            for k in range(len(cvs)):
                mi = jnp.minimum(mi, jnp.where(cvs[k] == mval, cis[k], bigi))
            midx = jnp.min(mi)
            out_vec = jnp.where(lane == t, midx, out_vec)
            for k in range(len(cvs)):
                cvs[k] = jnp.where(cis[k] == midx, jnp.inf, cvs[k])
        res_v[pl.ds(r * 16, 16)] = out_vec
        return carry

    jax.lax.fori_loop(0, _SC_ROWS, row_body, jnp.int32(0))
    pltpu.sync_copy(res_v, out_hbm.at[pl.ds(base * 16, _SC_ROWS * 16)])


def _sc_topk(d2):
    mesh = plsc.VectorSubcoreMesh(core_axis_name="c", subcore_axis_name="s")
    fn = functools.partial(
        pl.kernel,
        mesh=mesh,
        out_type=jax.ShapeDtypeStruct((_SC_N * 16,), jnp.int32),
        scratch_types=[
            pltpu.VMEM((_N,), jnp.float32),
            pltpu.VMEM((_CAP,), jnp.float32),
            pltpu.VMEM((_CAP,), jnp.int32),
            pltpu.VMEM((_SC_ROWS * 16,), jnp.int32),
        ],
        compiler_params=pltpu.CompilerParams(needs_layout_passes=False),
    )(_sc_topk_body)
    return fn(d2)


def kernel(images, img_coords, W):
    features = _project(images, W)
    d2_spatial = _d2_matrix(img_coords, _SC_N)
    nbr_sc16 = _sc_topk(d2_spatial).reshape(_SC_N, 16)
    nbr_latent = _knn_neighbors(features)
    nbr_sp_tail = _knn_neighbors(img_coords, block_offset=_SC_N // _RB)
    nbr_spatial = jnp.concatenate(
        [nbr_sc16[:, 1:_K + 1], nbr_sp_tail], axis=0)
    src = jnp.repeat(jnp.arange(_N, dtype=jnp.int32), _K)
    edge_spatial = jnp.stack([src, nbr_spatial.reshape(-1)], axis=0)
    edge_latent = jnp.stack([src, nbr_latent.reshape(-1)], axis=0)
    return features, edge_spatial, edge_latent, img_coords
